# Initial kernel scaffold; baseline (speedup 1.0000x reference)
import jax
import jax.numpy as jnp
from jax.experimental import pallas as pl


def kernel(embedding_features_per_residue, edge_index, edge_attr, batch, embedding_features_per_sequence, W1, b1, W2, b2, W3, b3, g1, be1, g2, be2, g3, be3, fc1_W, fc1_b, lin_W, lin_b):
    raise NotImplementedError("write your pallas kernel here")


# speedup vs baseline: 11.1362x; 11.1362x over previous
import functools

import jax
import jax.numpy as jnp
from jax import lax
from jax.experimental import pallas as pl
from jax.experimental.pallas import tpu as pltpu
from jax.experimental.pallas import tpu_sc as plsc

N = 10000
E = 320000
B = 16
D = 128
NCLS = 10
SEQ_D = 1280

NCORES = 2
NSUB = 16
NTILES = NCORES * NSUB
EB = 128
NB = 81
EROWS = NTILES * NB
EPAD = EROWS * EB
NPAD = 10240
NP = 10112
RPT = NP // NSUB
DCH = NPAD // NSUB

@functools.cache
def _mesh():
    return plsc.VectorSubcoreMesh(core_axis_name="c", subcore_axis_name="s",
                                  num_cores=NCORES, num_subcores=NSUB)


def _splat(v, i):
    idx = jnp.full((16, 1), i, jnp.int32)
    dnums = lax.GatherDimensionNumbers(
        offset_dims=(), collapsed_slice_dims=(0,), start_index_map=(0,))
    return lax.gather(v, idx, dnums, (1,),
                      mode=lax.GatherScatterMode.PROMISE_IN_BOUNDS)


def _deg_body(dst_hbm, ew_hbm, out_hbm, acc_sh, dstb, ewb, zb):
    c = lax.axis_index("c")
    s = lax.axis_index("s")
    wid = c * NSUB + s
    pltpu.sync_copy(dst_hbm.at[wid], dstb)
    pltpu.sync_copy(ew_hbm.at[wid], ewb)
    zero = jnp.zeros((16,), jnp.float32)

    def zrow(k, carry):
        zb[pl.ds(k * 16, 16)] = zero
        return carry

    lax.fori_loop(0, DCH // 16, zrow, 0)
    pltpu.sync_copy(zb, acc_sh.at[pl.ds(s * DCH, DCH)])
    plsc.subcore_barrier()

    def batch(j, carry):
        pltpu.sync_copy(ewb.at[j], acc_sh.at[dstb.at[j]], add=True)
        return carry

    lax.fori_loop(0, NB, batch, 0)
    plsc.subcore_barrier()
    pltpu.sync_copy(acc_sh.at[pl.ds(s * DCH, DCH)],
                    out_hbm.at[c, pl.ds(s * DCH, DCH)])


@functools.cache
def _deg_kernel():
    return pl.kernel(
        _deg_body,
        out_type=jax.ShapeDtypeStruct((NCORES, NPAD), jnp.float32),
        mesh=_mesh(),
        scratch_types=[
            pltpu.VMEM_SHARED((NPAD,), jnp.float32),
            pltpu.VMEM((NB, EB), jnp.int32),
            pltpu.VMEM((NB, EB), jnp.float32),
            pltpu.VMEM((DCH,), jnp.float32),
        ],
    )


def _scatter_body(hp_hbm, src_hbm, dst_hbm, ew_hbm, out_hbm,
                  acc_sh, srcb, dstb, ewb, rbuf, gsem):
    c = lax.axis_index("c")
    s = lax.axis_index("s")
    wid = c * NSUB + s
    pltpu.sync_copy(src_hbm.at[wid], srcb)
    pltpu.sync_copy(dst_hbm.at[wid], dstb)
    pltpu.sync_copy(ew_hbm.at[wid], ewb)

    zero = jnp.zeros((16,), jnp.float32)

    def zrow(r, carry):
        for f in range(8):
            rbuf[r, pl.ds(f * 16, 16)] = zero
        return carry

    lax.fori_loop(0, EB, zrow, 0)
    for k in range(RPT // EB):
        pltpu.sync_copy(rbuf, acc_sh.at[pl.ds(s * RPT + k * EB, EB)])
    rem = RPT - (RPT // EB) * EB
    if rem:
        pltpu.sync_copy(rbuf.at[pl.ds(0, rem)],
                        acc_sh.at[pl.ds(s * RPT + (RPT // EB) * EB, rem)])
    plsc.subcore_barrier()

    def batch(j, carry):
        pltpu.async_copy(hp_hbm.at[srcb.at[j]], rbuf, gsem).wait()

        def grp(g, carry2):
            nv = ewb[j, pl.ds(g * 16, 16)]
            for i in range(16):
                sp = _splat(nv, i)
                e = g * 16 + i
                for f in range(8):
                    rbuf[e, pl.ds(f * 16, 16)] = rbuf[e, pl.ds(f * 16, 16)] * sp
            return carry2

        lax.fori_loop(0, 8, grp, 0)
        pltpu.sync_copy(rbuf, acc_sh.at[dstb.at[j]], add=True)
        return carry

    lax.fori_loop(0, NB, batch, 0)
    plsc.subcore_barrier()
    pltpu.sync_copy(acc_sh.at[pl.ds(s * RPT, RPT)],
                    out_hbm.at[c, pl.ds(s * RPT, RPT)])


@functools.cache
def _scatter_kernel():
    return pl.kernel(
        _scatter_body,
        out_type=jax.ShapeDtypeStruct((NCORES, NP, D), jnp.float32),
        mesh=_mesh(),
        scratch_types=[
            pltpu.VMEM_SHARED((NP, D), jnp.float32),
            pltpu.VMEM((NB, EB), jnp.int32),
            pltpu.VMEM((NB, EB), jnp.int32),
            pltpu.VMEM((NB, EB), jnp.float32),
            pltpu.VMEM((EB, D), jnp.float32),
            pltpu.SemaphoreType.DMA,
        ],
    )


_BLK = 1000
_NBLK = N // _BLK


def _dot(a, b):
    return jnp.dot(a, b, preferred_element_type=jnp.float32,
                   precision=lax.Precision.HIGHEST)


def _disc_body(d0_ref, d1_ref, o_ref):
    o_ref[...] = lax.rsqrt(d0_ref[...] + d1_ref[...])


def _tc_disc(d0, d1):
    return pl.pallas_call(
        _disc_body,
        out_shape=jax.ShapeDtypeStruct((NPAD // D, D), jnp.float32),
    )(d0, d1)


def _prep_body(x_ref, w_ref, disc_ref, o_ref):
    o_ref[...] = disc_ref[...] * _dot(x_ref[...], w_ref[...])


def _tc_prep(x, w, disc):
    return pl.pallas_call(
        _prep_body,
        grid=(_NBLK,),
        in_specs=[
            pl.BlockSpec((_BLK, D), lambda i: (i, 0)),
            pl.BlockSpec((D, D), lambda i: (0, 0)),
            pl.BlockSpec((_BLK, 1), lambda i: (i, 0)),
        ],
        out_specs=pl.BlockSpec((_BLK, D), lambda i: (i, 0)),
        out_shape=jax.ShapeDtypeStruct((N, D), jnp.float32),
    )(x, w, disc)


def _post_body(a0_ref, a1_ref, disc_ref, b_ref, t_ref, st_ref, sacc, *, relu):
    i = pl.program_id(0)
    t = disc_ref[...] * (a0_ref[...] + a1_ref[...]) + b_ref[...]
    if relu:
        t = jnp.maximum(t, 0.0)
    t_ref[...] = t

    @pl.when(i == 0)
    def _():
        sacc[...] = jnp.zeros_like(sacc)

    sacc[0:1, :] += jnp.sum(t, axis=0, keepdims=True)
    sacc[1:2, :] += jnp.sum(t * t, axis=0, keepdims=True)
    st_ref[...] = sacc[...]


def _tc_post(a0, a1, disc, bias, relu):
    return pl.pallas_call(
        functools.partial(_post_body, relu=relu),
        grid=(_NBLK,),
        in_specs=[
            pl.BlockSpec((_BLK, D), lambda i: (i, 0)),
            pl.BlockSpec((_BLK, D), lambda i: (i, 0)),
            pl.BlockSpec((_BLK, 1), lambda i: (i, 0)),
            pl.BlockSpec((1, D), lambda i: (0, 0)),
        ],
        out_specs=[
            pl.BlockSpec((_BLK, D), lambda i: (i, 0)),
            pl.BlockSpec((2, D), lambda i: (0, 0)),
        ],
        out_shape=[
            jax.ShapeDtypeStruct((N, D), jnp.float32),
            jax.ShapeDtypeStruct((2, D), jnp.float32),
        ],
        scratch_shapes=[pltpu.VMEM((2, D), jnp.float32)],
    )(a0, a1, disc, bias)


def _bn(t, st_ref, g_ref, be_ref):
    mu = st_ref[0:1, :] * (1.0 / N)
    var = st_ref[1:2, :] * (1.0 / N) - mu * mu
    return (t - mu) * lax.rsqrt(var + 1e-5) * g_ref[...] + be_ref[...]


def _bnmm_body(t_ref, st_ref, g_ref, be_ref, w_ref, disc_ref, o_ref):
    xn = _bn(t_ref[...], st_ref, g_ref, be_ref)
    o_ref[...] = disc_ref[...] * _dot(xn, w_ref[...])


def _tc_bnmm(t, st, g, be, w, disc):
    return pl.pallas_call(
        _bnmm_body,
        grid=(_NBLK,),
        in_specs=[
            pl.BlockSpec((_BLK, D), lambda i: (i, 0)),
            pl.BlockSpec((2, D), lambda i: (0, 0)),
            pl.BlockSpec((1, D), lambda i: (0, 0)),
            pl.BlockSpec((1, D), lambda i: (0, 0)),
            pl.BlockSpec((D, D), lambda i: (0, 0)),
            pl.BlockSpec((_BLK, 1), lambda i: (i, 0)),
        ],
        out_specs=pl.BlockSpec((_BLK, D), lambda i: (i, 0)),
        out_shape=jax.ShapeDtypeStruct((N, D), jnp.float32),
    )(t, st, g, be, w, disc)


def _final_body(t_ref, st_ref, g_ref, be_ref, bat_ref, seq_ref, fw_ref,
                fb_ref, lw_ref, lb_ref, o_ref, pool_s, cnt_s):
    i = pl.program_id(0)

    @pl.when(i == 0)
    def _():
        pool_s[...] = jnp.zeros_like(pool_s)
        for cc in range(B):
            cnt_s[0, cc] = 0.0

    xn = _bn(t_ref[...], st_ref, g_ref, be_ref)
    bat = bat_ref[0]
    for cc in range(B):
        m = (bat == cc).astype(jnp.float32)
        pool_s[cc:cc + 1, :] += _dot(m, xn)
        cnt_s[0, cc] += jnp.sum(m)

    @pl.when(i == _NBLK - 1)
    def _():
        for cc in range(B):
            inv = 1.0 / jnp.maximum(cnt_s[0, cc], 1.0)
            pool_s[cc:cc + 1, :] *= inv
        z = pool_s[...] + _dot(seq_ref[...], fw_ref[...]) + fb_ref[...]
        o_ref[...] = jax.nn.sigmoid(_dot(z, lw_ref[...]) + lb_ref[...])


def _tc_final(t, st, g, be, bat3d, seq, fw, fb, lw, lb):
    return pl.pallas_call(
        _final_body,
        grid=(_NBLK,),
        in_specs=[
            pl.BlockSpec((_BLK, D), lambda i: (i, 0)),
            pl.BlockSpec((2, D), lambda i: (0, 0)),
            pl.BlockSpec((1, D), lambda i: (0, 0)),
            pl.BlockSpec((1, D), lambda i: (0, 0)),
            pl.BlockSpec((1, 1, _BLK), lambda i: (i, 0, 0)),
            pl.BlockSpec((B, SEQ_D), lambda i: (0, 0)),
            pl.BlockSpec((SEQ_D, D), lambda i: (0, 0)),
            pl.BlockSpec((1, D), lambda i: (0, 0)),
            pl.BlockSpec((D, D), lambda i: (0, 0)),
            pl.BlockSpec((1, D), lambda i: (0, 0)),
        ],
        out_specs=pl.BlockSpec((B, D), lambda i: (0, 0)),
        out_shape=jax.ShapeDtypeStruct((B, D), jnp.float32),
        scratch_shapes=[
            pltpu.VMEM((B, D), jnp.float32),
            pltpu.SMEM((1, B), jnp.float32),
        ],
    )(t, st, g, be, bat3d, seq, fw, fb, lw, lb)


def kernel(embedding_features_per_residue, edge_index, edge_attr, batch,
           embedding_features_per_sequence, W1, b1, W2, b2, W3, b3,
           g1, be1, g2, be2, g3, be3, fc1_W, fc1_b, lin_W, lin_b):
    x = embedding_features_per_residue
    src = edge_index[0]
    dst = edge_index[1]
    ew = edge_attr[:, 0]

    loop = jnp.arange(N, dtype=jnp.int32)
    padi = jnp.zeros((EPAD - E - N,), jnp.int32)
    srcA = jnp.concatenate([src, loop, padi]).reshape(NTILES, NB, EB)
    dstA = jnp.concatenate([dst, loop, padi]).reshape(NTILES, NB, EB)
    ewA = jnp.concatenate([ew, jnp.ones((N,), jnp.float32),
                           jnp.zeros((EPAD - E - N,), jnp.float32)]
                          ).reshape(NTILES, NB, EB)

    dega = _deg_kernel()(dstA, ewA)
    disc80 = _tc_disc(dega[0].reshape(NPAD // D, D),
                      dega[1].reshape(NPAD // D, D))
    disc = disc80.reshape(NPAD, 1)[:N]

    b1r, b2r, b3r = (v.reshape(1, D) for v in (b1, b2, b3))
    g1r, g2r, g3r = (v.reshape(1, D) for v in (g1, g2, g3))
    be1r, be2r, be3r = (v.reshape(1, D) for v in (be1, be2, be3))

    h = _tc_prep(x, W1, disc)

    acc = _scatter_kernel()(h, srcA, dstA, ewA)
    t, st = _tc_post(acc[0, :N], acc[1, :N], disc, b1r, relu=True)
    h = _tc_bnmm(t, st, g1r, be1r, W2, disc)

    acc = _scatter_kernel()(h, srcA, dstA, ewA)
    t, st = _tc_post(acc[0, :N], acc[1, :N], disc, b2r, relu=True)
    h = _tc_bnmm(t, st, g2r, be2r, W3, disc)

    acc = _scatter_kernel()(h, srcA, dstA, ewA)
    t, st = _tc_post(acc[0, :N], acc[1, :N], disc, b3r, relu=False)

    bat3d = batch.reshape(_NBLK, 1, _BLK)
    lwp = jnp.zeros((D, D), jnp.float32).at[:, :NCLS].set(lin_W)
    lbp = jnp.zeros((1, D), jnp.float32).at[0, :NCLS].set(lin_b)
    out = _tc_final(t, st, g3r, be3r, bat3d,
                    embedding_features_per_sequence, fc1_W,
                    fc1_b.reshape(1, D), lwp, lbp)
    return out[:, :NCLS]


# speedup vs baseline: 14.7081x; 1.3207x over previous
import functools

import jax
import jax.numpy as jnp
from jax import lax
from jax.experimental import pallas as pl
from jax.experimental.pallas import tpu as pltpu
from jax.experimental.pallas import tpu_sc as plsc

N = 10000
E = 320000
B = 16
D = 128
NCLS = 10
SEQ_D = 1280

NCORES = 2
NSUB = 16
NTILES = NCORES * NSUB
EB = 128
NB = 81
EROWS = NTILES * NB
EPAD = EROWS * EB
NPAD = 10240
NP = 10112
RPT = NP // NSUB
DCH = NPAD // NSUB

@functools.cache
def _mesh():
    return plsc.VectorSubcoreMesh(core_axis_name="c", subcore_axis_name="s",
                                  num_cores=NCORES, num_subcores=NSUB)


def _splat(v, i):
    idx = jnp.full((16, 1), i, jnp.int32)
    dnums = lax.GatherDimensionNumbers(
        offset_dims=(), collapsed_slice_dims=(0,), start_index_map=(0,))
    return lax.gather(v, idx, dnums, (1,),
                      mode=lax.GatherScatterMode.PROMISE_IN_BOUNDS)


def _deg_body(dst_hbm, ew_hbm, out_hbm, acc_sh, dstb, ewb, zb):
    c = lax.axis_index("c")
    s = lax.axis_index("s")
    wid = c * NSUB + s
    pltpu.sync_copy(dst_hbm.at[wid], dstb)
    pltpu.sync_copy(ew_hbm.at[wid], ewb)
    zero = jnp.zeros((16,), jnp.float32)

    def zrow(k, carry):
        zb[pl.ds(k * 16, 16)] = zero
        return carry

    lax.fori_loop(0, DCH // 16, zrow, 0)
    pltpu.sync_copy(zb, acc_sh.at[pl.ds(s * DCH, DCH)])
    plsc.subcore_barrier()

    def batch(j, carry):
        pltpu.sync_copy(ewb.at[j], acc_sh.at[dstb.at[j]], add=True)
        return carry

    lax.fori_loop(0, NB, batch, 0)
    plsc.subcore_barrier()
    pltpu.sync_copy(acc_sh.at[pl.ds(s * DCH, DCH)],
                    out_hbm.at[c, pl.ds(s * DCH, DCH)])


@functools.cache
def _deg_kernel():
    return pl.kernel(
        _deg_body,
        out_type=jax.ShapeDtypeStruct((NCORES, NPAD), jnp.float32),
        mesh=_mesh(),
        scratch_types=[
            pltpu.VMEM_SHARED((NPAD,), jnp.float32),
            pltpu.VMEM((NB, EB), jnp.int32),
            pltpu.VMEM((NB, EB), jnp.float32),
            pltpu.VMEM((DCH,), jnp.float32),
        ],
    )


def _scatter_body(hp_hbm, src_hbm, ew_hbm, dst_hbm, out_hbm,
                  acc_sh, dstb, ring, ringw, rbuf0, rbuf1, gsem0, gsem1, isem):
    c = lax.axis_index("c")
    s = lax.axis_index("s")
    wid = c * NSUB + s
    pltpu.sync_copy(dst_hbm.at[wid], dstb)

    zero = jnp.zeros((16,), jnp.float32)

    def zrow(r, carry):
        for f in range(8):
            rbuf0[r, pl.ds(f * 16, 16)] = zero
        return carry

    lax.fori_loop(0, EB, zrow, 0)
    for k in range(RPT // EB):
        pltpu.sync_copy(rbuf0, acc_sh.at[pl.ds(s * RPT + k * EB, EB)])
    rem = RPT - (RPT // EB) * EB
    if rem:
        pltpu.sync_copy(rbuf0.at[pl.ds(0, rem)],
                        acc_sh.at[pl.ds(s * RPT + (RPT // EB) * EB, rem)])
    plsc.subcore_barrier()

    pltpu.sync_copy(src_hbm.at[wid, 0], ring.at[0])
    pltpu.sync_copy(ew_hbm.at[wid, 0], ringw.at[0])
    pltpu.async_copy(hp_hbm.at[ring.at[0]], rbuf0, gsem0)
    pltpu.async_copy(src_hbm.at[wid, 1], ring.at[1], isem)
    pltpu.async_copy(ew_hbm.at[wid, 1], ringw.at[1], isem)

    def scale(jp, rbuf):
        def grp(g, carry2):
            nv = ringw[jp, pl.ds(g * 16, 16)]
            for i in range(16):
                sp = _splat(nv, i)
                e = g * 16 + i
                for f in range(8):
                    rbuf[e, pl.ds(f * 16, 16)] = rbuf[e, pl.ds(f * 16, 16)] * sp
            return carry2

        lax.fori_loop(0, EB // 16, grp, 0)

    def step(j, rbuf, gsem, obuf, osem):
        p = j % 2

        @pl.when(j + 1 < NB)
        def _():
            pltpu.make_async_copy(src_hbm.at[wid, j + 1], ring.at[1 - p],
                                  isem).wait()
            pltpu.make_async_copy(ew_hbm.at[wid, j + 1], ringw.at[1 - p],
                                  isem).wait()
            pltpu.async_copy(hp_hbm.at[ring.at[1 - p]], obuf, osem)

        scale(p, rbuf)

        @pl.when(j + 2 < NB)
        def _():
            pltpu.async_copy(src_hbm.at[wid, j + 2], ring.at[p], isem)
            pltpu.async_copy(ew_hbm.at[wid, j + 2], ringw.at[p], isem)

        pltpu.sync_copy(rbuf, acc_sh.at[dstb.at[j]], add=True)

    def pair(jp, carry):
        j0 = jp * 2
        pltpu.make_async_copy(hp_hbm.at[ring.at[0]], rbuf0, gsem0).wait()
        step(j0, rbuf0, gsem0, rbuf1, gsem1)
        pltpu.make_async_copy(hp_hbm.at[ring.at[1]], rbuf1, gsem1).wait()
        step(j0 + 1, rbuf1, gsem1, rbuf0, gsem0)
        return carry

    lax.fori_loop(0, NB // 2, pair, 0)

    pltpu.make_async_copy(hp_hbm.at[ring.at[0]], rbuf0, gsem0).wait()
    scale((NB - 1) % 2, rbuf0)
    pltpu.sync_copy(rbuf0, acc_sh.at[dstb.at[NB - 1]], add=True)

    plsc.subcore_barrier()
    pltpu.sync_copy(acc_sh.at[pl.ds(s * RPT, RPT)],
                    out_hbm.at[c, pl.ds(s * RPT, RPT)])


@functools.cache
def _scatter_kernel():
    return pl.kernel(
        _scatter_body,
        out_type=jax.ShapeDtypeStruct((NCORES, NP, D), jnp.float32),
        mesh=_mesh(),
        scratch_types=[
            pltpu.VMEM_SHARED((NP, D), jnp.float32),
            pltpu.VMEM((NB, EB), jnp.int32),
            pltpu.VMEM((2, EB), jnp.int32),
            pltpu.VMEM((2, EB), jnp.float32),
            pltpu.VMEM((EB, D), jnp.float32),
            pltpu.VMEM((EB, D), jnp.float32),
            pltpu.SemaphoreType.DMA,
            pltpu.SemaphoreType.DMA,
            pltpu.SemaphoreType.DMA,
        ],
    )


_BLK = 1000
_NBLK = N // _BLK


def _dot(a, b):
    return jnp.dot(a, b, preferred_element_type=jnp.float32,
                   precision=lax.Precision.HIGHEST)


def _disc_body(d0_ref, d1_ref, o_ref):
    o_ref[...] = lax.rsqrt(d0_ref[...] + d1_ref[...])


def _tc_disc(d0, d1):
    return pl.pallas_call(
        _disc_body,
        out_shape=jax.ShapeDtypeStruct((NPAD // D, D), jnp.float32),
    )(d0, d1)


def _prep_body(x_ref, w_ref, disc_ref, o_ref):
    o_ref[...] = disc_ref[...] * _dot(x_ref[...], w_ref[...])


def _tc_prep(x, w, disc):
    return pl.pallas_call(
        _prep_body,
        grid=(_NBLK,),
        in_specs=[
            pl.BlockSpec((_BLK, D), lambda i: (i, 0)),
            pl.BlockSpec((D, D), lambda i: (0, 0)),
            pl.BlockSpec((_BLK, 1), lambda i: (i, 0)),
        ],
        out_specs=pl.BlockSpec((_BLK, D), lambda i: (i, 0)),
        out_shape=jax.ShapeDtypeStruct((N, D), jnp.float32),
    )(x, w, disc)


def _post_body(a0_ref, a1_ref, disc_ref, b_ref, t_ref, st_ref, sacc, *, relu):
    i = pl.program_id(0)
    t = disc_ref[...] * (a0_ref[...] + a1_ref[...]) + b_ref[...]
    if relu:
        t = jnp.maximum(t, 0.0)
    t_ref[...] = t

    @pl.when(i == 0)
    def _():
        sacc[...] = jnp.zeros_like(sacc)

    sacc[0:1, :] += jnp.sum(t, axis=0, keepdims=True)
    sacc[1:2, :] += jnp.sum(t * t, axis=0, keepdims=True)
    st_ref[...] = sacc[...]


def _tc_post(a0, a1, disc, bias, relu):
    return pl.pallas_call(
        functools.partial(_post_body, relu=relu),
        grid=(_NBLK,),
        in_specs=[
            pl.BlockSpec((_BLK, D), lambda i: (i, 0)),
            pl.BlockSpec((_BLK, D), lambda i: (i, 0)),
            pl.BlockSpec((_BLK, 1), lambda i: (i, 0)),
            pl.BlockSpec((1, D), lambda i: (0, 0)),
        ],
        out_specs=[
            pl.BlockSpec((_BLK, D), lambda i: (i, 0)),
            pl.BlockSpec((2, D), lambda i: (0, 0)),
        ],
        out_shape=[
            jax.ShapeDtypeStruct((N, D), jnp.float32),
            jax.ShapeDtypeStruct((2, D), jnp.float32),
        ],
        scratch_shapes=[pltpu.VMEM((2, D), jnp.float32)],
    )(a0, a1, disc, bias)


def _bn(t, st_ref, g_ref, be_ref):
    mu = st_ref[0:1, :] * (1.0 / N)
    var = st_ref[1:2, :] * (1.0 / N) - mu * mu
    return (t - mu) * lax.rsqrt(var + 1e-5) * g_ref[...] + be_ref[...]


def _bnmm_body(t_ref, st_ref, g_ref, be_ref, w_ref, disc_ref, o_ref):
    xn = _bn(t_ref[...], st_ref, g_ref, be_ref)
    o_ref[...] = disc_ref[...] * _dot(xn, w_ref[...])


def _tc_bnmm(t, st, g, be, w, disc):
    return pl.pallas_call(
        _bnmm_body,
        grid=(_NBLK,),
        in_specs=[
            pl.BlockSpec((_BLK, D), lambda i: (i, 0)),
            pl.BlockSpec((2, D), lambda i: (0, 0)),
            pl.BlockSpec((1, D), lambda i: (0, 0)),
            pl.BlockSpec((1, D), lambda i: (0, 0)),
            pl.BlockSpec((D, D), lambda i: (0, 0)),
            pl.BlockSpec((_BLK, 1), lambda i: (i, 0)),
        ],
        out_specs=pl.BlockSpec((_BLK, D), lambda i: (i, 0)),
        out_shape=jax.ShapeDtypeStruct((N, D), jnp.float32),
    )(t, st, g, be, w, disc)


def _final_body(t_ref, st_ref, g_ref, be_ref, bat_ref, seq_ref, fw_ref,
                fb_ref, lw_ref, lb_ref, o_ref, pool_s, cnt_s):
    i = pl.program_id(0)

    @pl.when(i == 0)
    def _():
        pool_s[...] = jnp.zeros_like(pool_s)
        for cc in range(B):
            cnt_s[0, cc] = 0.0

    xn = _bn(t_ref[...], st_ref, g_ref, be_ref)
    bat = bat_ref[0]
    for cc in range(B):
        m = (bat == cc).astype(jnp.float32)
        pool_s[cc:cc + 1, :] += _dot(m, xn)
        cnt_s[0, cc] += jnp.sum(m)

    @pl.when(i == _NBLK - 1)
    def _():
        for cc in range(B):
            inv = 1.0 / jnp.maximum(cnt_s[0, cc], 1.0)
            pool_s[cc:cc + 1, :] *= inv
        z = pool_s[...] + _dot(seq_ref[...], fw_ref[...]) + fb_ref[...]
        o_ref[...] = jax.nn.sigmoid(_dot(z, lw_ref[...]) + lb_ref[...])


def _tc_final(t, st, g, be, bat3d, seq, fw, fb, lw, lb):
    return pl.pallas_call(
        _final_body,
        grid=(_NBLK,),
        in_specs=[
            pl.BlockSpec((_BLK, D), lambda i: (i, 0)),
            pl.BlockSpec((2, D), lambda i: (0, 0)),
            pl.BlockSpec((1, D), lambda i: (0, 0)),
            pl.BlockSpec((1, D), lambda i: (0, 0)),
            pl.BlockSpec((1, 1, _BLK), lambda i: (i, 0, 0)),
            pl.BlockSpec((B, SEQ_D), lambda i: (0, 0)),
            pl.BlockSpec((SEQ_D, D), lambda i: (0, 0)),
            pl.BlockSpec((1, D), lambda i: (0, 0)),
            pl.BlockSpec((D, D), lambda i: (0, 0)),
            pl.BlockSpec((1, D), lambda i: (0, 0)),
        ],
        out_specs=pl.BlockSpec((B, D), lambda i: (0, 0)),
        out_shape=jax.ShapeDtypeStruct((B, D), jnp.float32),
        scratch_shapes=[
            pltpu.VMEM((B, D), jnp.float32),
            pltpu.SMEM((1, B), jnp.float32),
        ],
    )(t, st, g, be, bat3d, seq, fw, fb, lw, lb)


def kernel(embedding_features_per_residue, edge_index, edge_attr, batch,
           embedding_features_per_sequence, W1, b1, W2, b2, W3, b3,
           g1, be1, g2, be2, g3, be3, fc1_W, fc1_b, lin_W, lin_b):
    x = embedding_features_per_residue
    src = edge_index[0]
    dst = edge_index[1]
    ew = edge_attr[:, 0]

    loop = jnp.arange(N, dtype=jnp.int32)
    padi = jnp.zeros((EPAD - E - N,), jnp.int32)
    srcA = jnp.concatenate([src, loop, padi]).reshape(NTILES, NB, EB)
    dstA = jnp.concatenate([dst, loop, padi]).reshape(NTILES, NB, EB)
    ewA = jnp.concatenate([ew, jnp.ones((N,), jnp.float32),
                           jnp.zeros((EPAD - E - N,), jnp.float32)]
                          ).reshape(NTILES, NB, EB)

    dega = _deg_kernel()(dstA, ewA)
    disc80 = _tc_disc(dega[0].reshape(NPAD // D, D),
                      dega[1].reshape(NPAD // D, D))
    disc = disc80.reshape(NPAD, 1)[:N]

    b1r, b2r, b3r = (v.reshape(1, D) for v in (b1, b2, b3))
    g1r, g2r, g3r = (v.reshape(1, D) for v in (g1, g2, g3))
    be1r, be2r, be3r = (v.reshape(1, D) for v in (be1, be2, be3))

    h = _tc_prep(x, W1, disc)

    acc = _scatter_kernel()(h, srcA, ewA, dstA)
    t, st = _tc_post(acc[0, :N], acc[1, :N], disc, b1r, relu=True)
    h = _tc_bnmm(t, st, g1r, be1r, W2, disc)

    acc = _scatter_kernel()(h, srcA, ewA, dstA)
    t, st = _tc_post(acc[0, :N], acc[1, :N], disc, b2r, relu=True)
    h = _tc_bnmm(t, st, g2r, be2r, W3, disc)

    acc = _scatter_kernel()(h, srcA, ewA, dstA)
    t, st = _tc_post(acc[0, :N], acc[1, :N], disc, b3r, relu=False)

    bat3d = batch.reshape(_NBLK, 1, _BLK)
    lwp = jnp.zeros((D, D), jnp.float32).at[:, :NCLS].set(lin_W)
    lbp = jnp.zeros((1, D), jnp.float32).at[0, :NCLS].set(lin_b)
    out = _tc_final(t, st, g3r, be3r, bat3d,
                    embedding_features_per_sequence, fc1_W,
                    fc1_b.reshape(1, D), lwp, lbp)
    return out[:, :NCLS]
